# baseline (speedup 1.0000x reference)
import jax
import jax.numpy as jnp
from jax import lax
from jax.experimental import pallas as pl
from jax.experimental.pallas import tpu as pltpu
from jax.experimental.pallas import tpu_sc as plsc

N = 10000
E = 320000
D = 128

NC = 2
NS = 16
NW = NC * NS
CH = 128
NCH = 80
EPW = NCH * CH
E_PAD = EPW * NW
ACC_ROWS = N + 8
ST = 640
PH = NCH // 2

_MB = 1000


def _agg_body(src_hbm, dst_hbm, m_hbm, out_hbm,
              sstage, dstage, sv0, sv1, dv0, dv1, r0, r1, acc,
              semi, sg0, sg1, ss0, ss1):
    srcv = [sv0, sv1]
    dstv = [dv0, dv1]
    rows = [r0, r1]
    semg = [sg0, sg1]
    sems = [ss0, ss1]
    c = lax.axis_index("c")
    s = lax.axis_index("s")
    w = c * NS + s

    ebase = pl.multiple_of(w * EPW, 8)
    pltpu.async_copy(src_hbm.at[pl.ds(ebase, EPW)], sstage, semi)

    zero16 = jnp.zeros((16,), jnp.float32)

    def _zfill(i, carry):
        for j in range(8):
            r0[i, pl.ds(j * 16, 16)] = zero16
        return carry

    lax.fori_loop(0, CH, _zfill, 0)
    ofs = pl.multiple_of(s * ST, 8)

    @pl.when(s < 15)
    def _():
        for k in range(ST // CH):
            pltpu.sync_copy(r0, acc.at[pl.ds(ofs + k * CH, CH), :])

    @pl.when(s == 15)
    def _():
        for k in range(3):
            pltpu.sync_copy(r0, acc.at[pl.ds(15 * ST + k * CH, CH), :])
        pltpu.sync_copy(r0.at[pl.ds(0, 24), :],
                        acc.at[pl.ds(15 * ST + 3 * CH, 24), :])

    pltpu.make_async_copy(src_hbm.at[pl.ds(0, EPW)], sstage, semi).wait()
    plsc.subcore_barrier()

    def copy_idx(gg, gl, b):
        for k in range(8):
            srcv[b][pl.ds(k * 16, 16)] = sstage[pl.ds(gg * CH + k * 16, 16)]
            dstv[b][pl.ds(k * 16, 16)] = dstage[pl.ds(gl * CH + k * 16, 16)]

    def gather(b):
        pltpu.async_copy(m_hbm.at[srcv[b]], rows[b], semg[b])

    def wait_gather(b):
        pltpu.make_async_copy(m_hbm.at[pl.ds(0, CH), :], rows[b],
                              semg[b]).wait()

    def scatter(b):
        pltpu.async_copy(rows[b], acc.at[dstv[b]], sems[b], add=True)

    def wait_scatter(b):
        pltpu.make_async_copy(rows[b], acc.at[pl.ds(0, CH), :],
                              sems[b]).wait()

    for p in range(2):
        base = pl.multiple_of(w * EPW + p * PH * CH, 8)
        pltpu.async_copy(dst_hbm.at[pl.ds(base, PH * CH)], dstage, semi)
        pltpu.make_async_copy(dst_hbm.at[pl.ds(0, PH * CH)], dstage,
                              semi).wait()
        g0 = p * PH
        copy_idx(g0, 0, 0)
        gather(0)
        wait_gather(0)
        scatter(0)
        copy_idx(g0 + 1, 1, 1)
        gather(1)

        def _pair(j, carry):
            wait_gather(1)
            scatter(1)
            wait_scatter(0)
            copy_idx(g0 + 2 * j + 2, 2 * j + 2, 0)
            gather(0)
            wait_gather(0)
            scatter(0)
            wait_scatter(1)
            copy_idx(g0 + 2 * j + 3, 2 * j + 3, 1)
            gather(1)
            return carry

        lax.fori_loop(0, PH // 2 - 1, _pair, 0)
        wait_gather(1)
        scatter(1)
        wait_scatter(0)
        wait_scatter(1)

    plsc.subcore_barrier()

    @pl.when(s < 15)
    def _():
        pltpu.sync_copy(acc.at[pl.ds(ofs, ST), :],
                        out_hbm.at[c, pl.ds(ofs, ST), :])

    @pl.when(s == 15)
    def _():
        pltpu.sync_copy(acc.at[pl.ds(15 * ST, N - 15 * ST), :],
                        out_hbm.at[c, pl.ds(15 * ST, N - 15 * ST), :])


@jax.jit
def _agg(src3, dst3, m):
    mesh = plsc.VectorSubcoreMesh(core_axis_name="c", subcore_axis_name="s")
    row_t = pltpu.VMEM((CH, D), jnp.float32)
    return pl.kernel(
        _agg_body,
        out_type=jax.ShapeDtypeStruct((NC, N, D), jnp.float32),
        mesh=mesh,
        scratch_types=[
            pltpu.VMEM((EPW,), jnp.int32),
            pltpu.VMEM((PH * CH,), jnp.int32),
            pltpu.VMEM((CH,), jnp.int32),
            pltpu.VMEM((CH,), jnp.int32),
            pltpu.VMEM((CH,), jnp.int32),
            pltpu.VMEM((CH,), jnp.int32),
            row_t, row_t,
            pltpu.VMEM_SHARED((ACC_ROWS, D), jnp.float32),
        ] + [pltpu.SemaphoreType.DMA] * 5,
    )(src3, dst3, m)


def _mm_body(x_ref, w_ref, o_ref):
    o_ref[...] = jnp.dot(x_ref[...], w_ref[...],
                         preferred_element_type=jnp.float32)


@jax.jit
def _mm(x, w):
    return pl.pallas_call(
        _mm_body,
        grid=(N // _MB,),
        in_specs=[
            pl.BlockSpec((_MB, D), lambda i: (i, 0)),
            pl.BlockSpec((D, D), lambda i: (0, 0)),
        ],
        out_specs=pl.BlockSpec((_MB, D), lambda i: (i, 0)),
        out_shape=jax.ShapeDtypeStruct((N, D), jnp.float32),
    )(x, w)


def _combine_mm_body(p_ref, b_ref, w_ref, o_ref):
    x = jnp.maximum(p_ref[0] + p_ref[1] + b_ref[...], 0.0)
    o_ref[...] = jnp.dot(x, w_ref[...], preferred_element_type=jnp.float32)


@jax.jit
def _combine_mm(p, b, w):
    return pl.pallas_call(
        _combine_mm_body,
        grid=(N // _MB,),
        in_specs=[
            pl.BlockSpec((NC, _MB, D), lambda i: (0, i, 0)),
            pl.BlockSpec((1, D), lambda i: (0, 0)),
            pl.BlockSpec((D, D), lambda i: (0, 0)),
        ],
        out_specs=pl.BlockSpec((_MB, D), lambda i: (i, 0)),
        out_shape=jax.ShapeDtypeStruct((N, D), jnp.float32),
    )(p, b, w)


def _head_body(p_ref, b2_ref, wh1_ref, bh1_ref, wh2_ref, bh2_ref,
               wc_ref, bc_ref, o_ref):
    x = jnp.maximum(p_ref[0] + p_ref[1] + b2_ref[...], 0.0)
    h = jnp.maximum(
        jnp.dot(x, wh1_ref[...], preferred_element_type=jnp.float32)
        + bh1_ref[...], 0.0)
    h = jnp.maximum(
        jnp.dot(h, wh2_ref[...], preferred_element_type=jnp.float32)
        + bh2_ref[...], 0.0)
    av = (jnp.dot(h, wc_ref[...], preferred_element_type=jnp.float32)
          + bc_ref[...])
    col = lax.broadcasted_iota(jnp.int32, av.shape, 1)
    adv_sum = jnp.sum(jnp.where(col < 5, av, 0.0), axis=1, keepdims=True)
    val = jnp.sum(jnp.where(col == 5, av, 0.0), axis=1, keepdims=True)
    o_ref[...] = val + av - adv_sum * (1.0 / 5.0)


@jax.jit
def _head(p, b2, wh1, bh1, wh2, bh2, wc, bc):
    return pl.pallas_call(
        _head_body,
        grid=(N // _MB,),
        in_specs=[
            pl.BlockSpec((NC, _MB, D), lambda i: (0, i, 0)),
            pl.BlockSpec((1, D), lambda i: (0, 0)),
            pl.BlockSpec((D, D), lambda i: (0, 0)),
            pl.BlockSpec((1, D), lambda i: (0, 0)),
            pl.BlockSpec((D, D), lambda i: (0, 0)),
            pl.BlockSpec((1, D), lambda i: (0, 0)),
            pl.BlockSpec((D, 8), lambda i: (0, 0)),
            pl.BlockSpec((1, 8), lambda i: (0, 0)),
        ],
        out_specs=pl.BlockSpec((_MB, 8), lambda i: (i, 0)),
        out_shape=jax.ShapeDtypeStruct((N, 8), jnp.float32),
    )(p, b2, wh1, bh1, wh2, bh2, wc, bc)


def kernel(edge_index, entity_embeddings, W1, b1, W2, b2,
           Wh1, bh1, Wh2, bh2, Wadv, badv, Wval, bval):
    pad = E_PAD - E
    src = jnp.concatenate([edge_index[0], jnp.zeros((pad,), jnp.int32)])
    dst = jnp.concatenate([edge_index[1], jnp.full((pad,), N, jnp.int32)])
    wc = jnp.concatenate([Wadv, Wval, jnp.zeros((D, 2), jnp.float32)], axis=1)
    bc = jnp.concatenate([badv, bval, jnp.zeros((2,), jnp.float32)])[None, :]

    m1 = _mm(entity_embeddings, W1)
    p1 = _agg(src, dst, m1)
    m2 = _combine_mm(p1, b1[None, :], W2)
    p2 = _agg(src, dst, m2)
    q8 = _head(p2, b2[None, :], Wh1, bh1[None, :], Wh2, bh2[None, :], wc, bc)
    return q8[:, :5]


# speedup vs baseline: 2.0030x; 2.0030x over previous
import jax
import jax.numpy as jnp
from jax import lax
from jax.experimental import pallas as pl
from jax.experimental.pallas import tpu as pltpu
from jax.experimental.pallas import tpu_sc as plsc

N = 10000
E = 320000
D = 128

NC = 2
NS = 16
NW = NC * NS
EPW = E // NW
CH = 128
NFULL = EPW // CH
REM = EPW - NFULL * CH
ST = 640
ST_LAST = N - 15 * ST

_MB = 1000


def _agg_body(src_hbm, dst_hbm, m_hbm, out_hbm,
              srcv, dstv, rows, srcr, dstr, rowsr, acc, sem):
    c = lax.axis_index("c")
    s = lax.axis_index("s")

    zero16 = jnp.zeros((16,), jnp.float32)

    def _zfill(i, carry):
        for j in range(8):
            rows[i, pl.ds(j * 16, 16)] = zero16
        return carry

    lax.fori_loop(0, CH, _zfill, 0)

    def _zfill_r(i, carry):
        for j in range(8):
            rowsr[i, pl.ds(j * 16, 16)] = zero16
        return carry

    lax.fori_loop(0, REM, _zfill_r, 0)
    ofs = pl.multiple_of(s * ST, 8)

    @pl.when(s < 15)
    def _():
        for k in range(ST // CH):
            pltpu.sync_copy(rows, acc.at[pl.ds(ofs + k * CH, CH), :])

    @pl.when(s == 15)
    def _():
        for k in range(3):
            pltpu.sync_copy(rows, acc.at[pl.ds(15 * ST + k * CH, CH), :])
        pltpu.sync_copy(rowsr, acc.at[pl.ds(15 * ST + 3 * CH, REM), :])

    plsc.subcore_barrier()

    ebase = c * (E // NC) + s * EPW

    def _chunk(i, carry):
        base = pl.multiple_of(ebase + i * CH, 8)
        pltpu.sync_copy(src_hbm.at[pl.ds(base, CH)], srcv)
        pltpu.sync_copy(dst_hbm.at[pl.ds(base, CH)], dstv)
        pltpu.async_copy(m_hbm.at[srcv], rows, sem).wait()
        pltpu.sync_copy(rows, acc.at[dstv], add=True)
        return carry

    lax.fori_loop(0, NFULL, _chunk, 0)

    base = pl.multiple_of(ebase + NFULL * CH, 8)
    pltpu.sync_copy(src_hbm.at[pl.ds(base, REM)], srcr)
    pltpu.sync_copy(dst_hbm.at[pl.ds(base, REM)], dstr)
    pltpu.async_copy(m_hbm.at[srcr], rowsr, sem).wait()
    pltpu.sync_copy(rowsr, acc.at[dstr], add=True)

    plsc.subcore_barrier()

    @pl.when(s < 15)
    def _():
        pltpu.sync_copy(acc.at[pl.ds(ofs, ST), :],
                        out_hbm.at[c, pl.ds(ofs, ST), :])

    @pl.when(s == 15)
    def _():
        pltpu.sync_copy(acc.at[pl.ds(15 * ST, ST_LAST), :],
                        out_hbm.at[c, pl.ds(15 * ST, ST_LAST), :])


@jax.jit
def _agg(src, dst, m):
    mesh = plsc.VectorSubcoreMesh(core_axis_name="c", subcore_axis_name="s")
    return pl.kernel(
        _agg_body,
        out_type=jax.ShapeDtypeStruct((NC, N, D), jnp.float32),
        mesh=mesh,
        scratch_types=[
            pltpu.VMEM((CH,), jnp.int32),
            pltpu.VMEM((CH,), jnp.int32),
            pltpu.VMEM((CH, D), jnp.float32),
            pltpu.VMEM((REM,), jnp.int32),
            pltpu.VMEM((REM,), jnp.int32),
            pltpu.VMEM((REM, D), jnp.float32),
            pltpu.VMEM_SHARED((N, D), jnp.float32),
            pltpu.SemaphoreType.DMA,
        ],
    )(src, dst, m)


def _mm_body(x_ref, w_ref, o_ref):
    o_ref[...] = jnp.dot(x_ref[...], w_ref[...],
                         preferred_element_type=jnp.float32)


@jax.jit
def _mm(x, w):
    return pl.pallas_call(
        _mm_body,
        grid=(N // _MB,),
        in_specs=[
            pl.BlockSpec((_MB, D), lambda i: (i, 0)),
            pl.BlockSpec((D, D), lambda i: (0, 0)),
        ],
        out_specs=pl.BlockSpec((_MB, D), lambda i: (i, 0)),
        out_shape=jax.ShapeDtypeStruct((N, D), jnp.float32),
    )(x, w)


def _combine_mm_body(p_ref, b_ref, w_ref, o_ref):
    x = jnp.maximum(p_ref[0] + p_ref[1] + b_ref[...], 0.0)
    o_ref[...] = jnp.dot(x, w_ref[...], preferred_element_type=jnp.float32)


@jax.jit
def _combine_mm(p, b, w):
    return pl.pallas_call(
        _combine_mm_body,
        grid=(N // _MB,),
        in_specs=[
            pl.BlockSpec((NC, _MB, D), lambda i: (0, i, 0)),
            pl.BlockSpec((1, D), lambda i: (0, 0)),
            pl.BlockSpec((D, D), lambda i: (0, 0)),
        ],
        out_specs=pl.BlockSpec((_MB, D), lambda i: (i, 0)),
        out_shape=jax.ShapeDtypeStruct((N, D), jnp.float32),
    )(p, b, w)


def _head_body(p_ref, b2_ref, wh1_ref, bh1_ref, wh2_ref, bh2_ref,
               wc_ref, bc_ref, o_ref):
    x = jnp.maximum(p_ref[0] + p_ref[1] + b2_ref[...], 0.0)
    h = jnp.maximum(
        jnp.dot(x, wh1_ref[...], preferred_element_type=jnp.float32)
        + bh1_ref[...], 0.0)
    h = jnp.maximum(
        jnp.dot(h, wh2_ref[...], preferred_element_type=jnp.float32)
        + bh2_ref[...], 0.0)
    av = (jnp.dot(h, wc_ref[...], preferred_element_type=jnp.float32)
          + bc_ref[...])
    col = lax.broadcasted_iota(jnp.int32, av.shape, 1)
    adv_sum = jnp.sum(jnp.where(col < 5, av, 0.0), axis=1, keepdims=True)
    val = jnp.sum(jnp.where(col == 5, av, 0.0), axis=1, keepdims=True)
    o_ref[...] = val + av - adv_sum * (1.0 / 5.0)


@jax.jit
def _head(p, b2, wh1, bh1, wh2, bh2, wc, bc):
    return pl.pallas_call(
        _head_body,
        grid=(N // _MB,),
        in_specs=[
            pl.BlockSpec((NC, _MB, D), lambda i: (0, i, 0)),
            pl.BlockSpec((1, D), lambda i: (0, 0)),
            pl.BlockSpec((D, D), lambda i: (0, 0)),
            pl.BlockSpec((1, D), lambda i: (0, 0)),
            pl.BlockSpec((D, D), lambda i: (0, 0)),
            pl.BlockSpec((1, D), lambda i: (0, 0)),
            pl.BlockSpec((D, 8), lambda i: (0, 0)),
            pl.BlockSpec((1, 8), lambda i: (0, 0)),
        ],
        out_specs=pl.BlockSpec((_MB, 8), lambda i: (i, 0)),
        out_shape=jax.ShapeDtypeStruct((N, 8), jnp.float32),
    )(p, b2, wh1, bh1, wh2, bh2, wc, bc)


def kernel(edge_index, entity_embeddings, W1, b1, W2, b2,
           Wh1, bh1, Wh2, bh2, Wadv, badv, Wval, bval):
    src = edge_index[0]
    dst = edge_index[1]
    wc = jnp.concatenate([Wadv, Wval, jnp.zeros((D, 2), jnp.float32)], axis=1)
    bc = jnp.concatenate([badv, bval, jnp.zeros((2,), jnp.float32)])[None, :]

    m1 = _mm(entity_embeddings, W1)
    p1 = _agg(src, dst, m1)
    m2 = _combine_mm(p1, b1[None, :], W2)
    p2 = _agg(src, dst, m2)
    q8 = _head(p2, b2[None, :], Wh1, bh1[None, :], Wh2, bh2[None, :], wc, bc)
    return q8[:, :5]


# speedup vs baseline: 2.4226x; 1.2095x over previous
import jax
import jax.numpy as jnp
from jax import lax
from jax.experimental import pallas as pl
from jax.experimental.pallas import tpu as pltpu
from jax.experimental.pallas import tpu_sc as plsc

N = 10000
E = 320000
D = 128

NC = 2
NS = 16
NW = NC * NS
EPW = E // NW
CH = 128
NFULL = EPW // CH
REM = EPW - NFULL * CH
ST = 640
ST_LAST = N - 15 * ST

_MB = 1000


def _agg_body(src_hbm, dst_hbm, m_hbm, out_hbm,
              srcv, dv0, dv1, r0, r1, srcr, dstr, rowsr, acc,
              gsem, ss0, ss1):
    dstv = [dv0, dv1]
    rows = [r0, r1]
    ssem = [ss0, ss1]
    c = lax.axis_index("c")
    s = lax.axis_index("s")

    zero16 = jnp.zeros((16,), jnp.float32)

    def _zfill(i, carry):
        for j in range(8):
            r0[i, pl.ds(j * 16, 16)] = zero16
        return carry

    lax.fori_loop(0, CH, _zfill, 0)

    def _zfill_r(i, carry):
        for j in range(8):
            rowsr[i, pl.ds(j * 16, 16)] = zero16
        return carry

    lax.fori_loop(0, REM, _zfill_r, 0)
    ofs = pl.multiple_of(s * ST, 8)

    @pl.when(s < 15)
    def _():
        for k in range(ST // CH):
            pltpu.sync_copy(r0, acc.at[pl.ds(ofs + k * CH, CH), :])

    @pl.when(s == 15)
    def _():
        for k in range(3):
            pltpu.sync_copy(r0, acc.at[pl.ds(15 * ST + k * CH, CH), :])
        pltpu.sync_copy(rowsr, acc.at[pl.ds(15 * ST + 3 * CH, REM), :])

    plsc.subcore_barrier()

    ebase = c * (E // NC) + s * EPW

    def _do(i, b, with_wait):
        base = pl.multiple_of(ebase + i * CH, 8)
        pltpu.sync_copy(src_hbm.at[pl.ds(base, CH)], srcv)
        if with_wait:
            pltpu.make_async_copy(rows[b], acc.at[pl.ds(0, CH), :],
                                  ssem[b]).wait()
        pltpu.sync_copy(dst_hbm.at[pl.ds(base, CH)], dstv[b])
        pltpu.async_copy(m_hbm.at[srcv], rows[b], gsem).wait()
        pltpu.async_copy(rows[b], acc.at[dstv[b]], ssem[b], add=True)

    _do(0, 0, False)
    _do(1, 1, False)

    def _pair(j, carry):
        _do(2 * j, 0, True)
        _do(2 * j + 1, 1, True)
        return carry

    lax.fori_loop(1, NFULL // 2, _pair, 0)
    for b in range(2):
        pltpu.make_async_copy(rows[b], acc.at[pl.ds(0, CH), :],
                              ssem[b]).wait()

    base = pl.multiple_of(ebase + NFULL * CH, 8)
    pltpu.sync_copy(src_hbm.at[pl.ds(base, REM)], srcr)
    pltpu.sync_copy(dst_hbm.at[pl.ds(base, REM)], dstr)
    pltpu.async_copy(m_hbm.at[srcr], rowsr, gsem).wait()
    pltpu.sync_copy(rowsr, acc.at[dstr], add=True)

    plsc.subcore_barrier()

    @pl.when(s < 15)
    def _():
        pltpu.sync_copy(acc.at[pl.ds(ofs, ST), :],
                        out_hbm.at[c, pl.ds(ofs, ST), :])

    @pl.when(s == 15)
    def _():
        pltpu.sync_copy(acc.at[pl.ds(15 * ST, ST_LAST), :],
                        out_hbm.at[c, pl.ds(15 * ST, ST_LAST), :])


@jax.jit
def _agg(src, dst, m):
    mesh = plsc.VectorSubcoreMesh(core_axis_name="c", subcore_axis_name="s")
    return pl.kernel(
        _agg_body,
        out_type=jax.ShapeDtypeStruct((NC, N, D), jnp.float32),
        mesh=mesh,
        scratch_types=[
            pltpu.VMEM((CH,), jnp.int32),
            pltpu.VMEM((CH,), jnp.int32),
            pltpu.VMEM((CH,), jnp.int32),
            pltpu.VMEM((CH, D), jnp.float32),
            pltpu.VMEM((CH, D), jnp.float32),
            pltpu.VMEM((REM,), jnp.int32),
            pltpu.VMEM((REM,), jnp.int32),
            pltpu.VMEM((REM, D), jnp.float32),
            pltpu.VMEM_SHARED((N, D), jnp.float32),
        ] + [pltpu.SemaphoreType.DMA] * 3,
    )(src, dst, m)


def _mm_body(x_ref, w_ref, o_ref):
    o_ref[...] = jnp.dot(x_ref[...], w_ref[...],
                         preferred_element_type=jnp.float32)


@jax.jit
def _mm(x, w):
    return pl.pallas_call(
        _mm_body,
        grid=(N // _MB,),
        in_specs=[
            pl.BlockSpec((_MB, D), lambda i: (i, 0)),
            pl.BlockSpec((D, D), lambda i: (0, 0)),
        ],
        out_specs=pl.BlockSpec((_MB, D), lambda i: (i, 0)),
        out_shape=jax.ShapeDtypeStruct((N, D), jnp.float32),
    )(x, w)


def _combine_mm_body(p_ref, b_ref, w_ref, o_ref):
    x = jnp.maximum(p_ref[0] + p_ref[1] + b_ref[...], 0.0)
    o_ref[...] = jnp.dot(x, w_ref[...], preferred_element_type=jnp.float32)


@jax.jit
def _combine_mm(p, b, w):
    return pl.pallas_call(
        _combine_mm_body,
        grid=(N // _MB,),
        in_specs=[
            pl.BlockSpec((NC, _MB, D), lambda i: (0, i, 0)),
            pl.BlockSpec((1, D), lambda i: (0, 0)),
            pl.BlockSpec((D, D), lambda i: (0, 0)),
        ],
        out_specs=pl.BlockSpec((_MB, D), lambda i: (i, 0)),
        out_shape=jax.ShapeDtypeStruct((N, D), jnp.float32),
    )(p, b, w)


def _head_body(p_ref, b2_ref, wh1_ref, bh1_ref, wh2_ref, bh2_ref,
               wc_ref, bc_ref, o_ref):
    x = jnp.maximum(p_ref[0] + p_ref[1] + b2_ref[...], 0.0)
    h = jnp.maximum(
        jnp.dot(x, wh1_ref[...], preferred_element_type=jnp.float32)
        + bh1_ref[...], 0.0)
    h = jnp.maximum(
        jnp.dot(h, wh2_ref[...], preferred_element_type=jnp.float32)
        + bh2_ref[...], 0.0)
    av = (jnp.dot(h, wc_ref[...], preferred_element_type=jnp.float32)
          + bc_ref[...])
    col = lax.broadcasted_iota(jnp.int32, av.shape, 1)
    adv_sum = jnp.sum(jnp.where(col < 5, av, 0.0), axis=1, keepdims=True)
    val = jnp.sum(jnp.where(col == 5, av, 0.0), axis=1, keepdims=True)
    o_ref[...] = val + av - adv_sum * (1.0 / 5.0)


@jax.jit
def _head(p, b2, wh1, bh1, wh2, bh2, wc, bc):
    return pl.pallas_call(
        _head_body,
        grid=(N // _MB,),
        in_specs=[
            pl.BlockSpec((NC, _MB, D), lambda i: (0, i, 0)),
            pl.BlockSpec((1, D), lambda i: (0, 0)),
            pl.BlockSpec((D, D), lambda i: (0, 0)),
            pl.BlockSpec((1, D), lambda i: (0, 0)),
            pl.BlockSpec((D, D), lambda i: (0, 0)),
            pl.BlockSpec((1, D), lambda i: (0, 0)),
            pl.BlockSpec((D, 8), lambda i: (0, 0)),
            pl.BlockSpec((1, 8), lambda i: (0, 0)),
        ],
        out_specs=pl.BlockSpec((_MB, 8), lambda i: (i, 0)),
        out_shape=jax.ShapeDtypeStruct((N, 8), jnp.float32),
    )(p, b2, wh1, bh1, wh2, bh2, wc, bc)


def kernel(edge_index, entity_embeddings, W1, b1, W2, b2,
           Wh1, bh1, Wh2, bh2, Wadv, badv, Wval, bval):
    src = edge_index[0]
    dst = edge_index[1]
    wc = jnp.concatenate([Wadv, Wval, jnp.zeros((D, 2), jnp.float32)], axis=1)
    bc = jnp.concatenate([badv, bval, jnp.zeros((2,), jnp.float32)])[None, :]

    m1 = _mm(entity_embeddings, W1)
    p1 = _agg(src, dst, m1)
    m2 = _combine_mm(p1, b1[None, :], W2)
    p2 = _agg(src, dst, m2)
    q8 = _head(p2, b2[None, :], Wh1, bh1[None, :], Wh2, bh2[None, :], wc, bc)
    return q8[:, :5]


# speedup vs baseline: 3.6035x; 1.4875x over previous
import jax
import jax.numpy as jnp
from jax import lax
from jax.experimental import pallas as pl
from jax.experimental.pallas import tpu as pltpu
from jax.experimental.pallas import tpu_sc as plsc

N = 10000
E = 320000
D = 128

NC = 2
NS = 16
NW = NC * NS
EPW = E // NW
CH = 128
NFULL = EPW // CH
REM = EPW - NFULL * CH
ST = 640
ST_LAST = N - 15 * ST

_MB = 1000


def _agg_body(src_hbm, dst_hbm, m_hbm, out_hbm,
              sv0, sv1, dv0, dv1, r0, r1, srcr, dstr, rowsr, acc,
              gs0, gs1, ss0, ss1):
    srcv = [sv0, sv1]
    dstv = [dv0, dv1]
    rows = [r0, r1]
    gsem = [gs0, gs1]
    ssem = [ss0, ss1]
    c = lax.axis_index("c")
    s = lax.axis_index("s")

    zero16 = jnp.zeros((16,), jnp.float32)

    def _zfill(i, carry):
        for j in range(8):
            r0[i, pl.ds(j * 16, 16)] = zero16
        return carry

    lax.fori_loop(0, CH, _zfill, 0)

    def _zfill_r(i, carry):
        for j in range(8):
            rowsr[i, pl.ds(j * 16, 16)] = zero16
        return carry

    lax.fori_loop(0, REM, _zfill_r, 0)
    ofs = pl.multiple_of(s * ST, 8)

    @pl.when(s < 15)
    def _():
        for k in range(ST // CH):
            pltpu.sync_copy(r0, acc.at[pl.ds(ofs + k * CH, CH), :])

    @pl.when(s == 15)
    def _():
        for k in range(3):
            pltpu.sync_copy(r0, acc.at[pl.ds(15 * ST + k * CH, CH), :])
        pltpu.sync_copy(rowsr, acc.at[pl.ds(15 * ST + 3 * CH, REM), :])

    plsc.subcore_barrier()

    ebase = c * (E // NC) + s * EPW

    def fetch_src(i, b):
        base = pl.multiple_of(ebase + i * CH, 8)
        pltpu.sync_copy(src_hbm.at[pl.ds(base, CH)], srcv[b])

    def fetch_dst(i, b):
        base = pl.multiple_of(ebase + i * CH, 8)
        pltpu.sync_copy(dst_hbm.at[pl.ds(base, CH)], dstv[b])

    def wait_scatter(b):
        pltpu.make_async_copy(rows[b], acc.at[pl.ds(0, CH), :],
                              ssem[b]).wait()

    def gather(b):
        pltpu.async_copy(m_hbm.at[srcv[b]], rows[b], gsem[b])

    def wait_gather(b):
        pltpu.make_async_copy(m_hbm.at[pl.ds(0, CH), :], rows[b],
                              gsem[b]).wait()

    def scatter(b):
        pltpu.async_copy(rows[b], acc.at[dstv[b]], ssem[b], add=True)

    fetch_src(0, 0)
    fetch_dst(0, 0)
    gather(0)
    fetch_src(1, 1)
    fetch_dst(1, 1)
    gather(1)
    wait_gather(0)
    scatter(0)

    def _do(i, b):
        fetch_src(i, b)
        wait_scatter(b)
        fetch_dst(i, b)
        gather(b)
        wait_gather(1 - b)
        scatter(1 - b)

    def _pair(j, carry):
        _do(2 * j, 0)
        _do(2 * j + 1, 1)
        return carry

    lax.fori_loop(1, NFULL // 2, _pair, 0)
    wait_gather(1)
    scatter(1)
    for b in range(2):
        wait_scatter(b)

    base = pl.multiple_of(ebase + NFULL * CH, 8)
    pltpu.sync_copy(src_hbm.at[pl.ds(base, REM)], srcr)
    pltpu.sync_copy(dst_hbm.at[pl.ds(base, REM)], dstr)
    pltpu.async_copy(m_hbm.at[srcr], rowsr, gs0).wait()
    pltpu.sync_copy(rowsr, acc.at[dstr], add=True)

    plsc.subcore_barrier()

    @pl.when(s < 15)
    def _():
        pltpu.sync_copy(acc.at[pl.ds(ofs, ST), :],
                        out_hbm.at[c, pl.ds(ofs, ST), :])

    @pl.when(s == 15)
    def _():
        pltpu.sync_copy(acc.at[pl.ds(15 * ST, ST_LAST), :],
                        out_hbm.at[c, pl.ds(15 * ST, ST_LAST), :])


@jax.jit
def _agg(src, dst, m):
    mesh = plsc.VectorSubcoreMesh(core_axis_name="c", subcore_axis_name="s")
    return pl.kernel(
        _agg_body,
        out_type=jax.ShapeDtypeStruct((NC, N, D), jnp.float32),
        mesh=mesh,
        scratch_types=[
            pltpu.VMEM((CH,), jnp.int32),
            pltpu.VMEM((CH,), jnp.int32),
            pltpu.VMEM((CH,), jnp.int32),
            pltpu.VMEM((CH,), jnp.int32),
            pltpu.VMEM((CH, D), jnp.float32),
            pltpu.VMEM((CH, D), jnp.float32),
            pltpu.VMEM((REM,), jnp.int32),
            pltpu.VMEM((REM,), jnp.int32),
            pltpu.VMEM((REM, D), jnp.float32),
            pltpu.VMEM_SHARED((N, D), jnp.float32),
        ] + [pltpu.SemaphoreType.DMA] * 4,
    )(src, dst, m)


def _mm_body(x_ref, w_ref, o_ref):
    o_ref[...] = jnp.dot(x_ref[...], w_ref[...],
                         preferred_element_type=jnp.float32)


@jax.jit
def _mm(x, w):
    return pl.pallas_call(
        _mm_body,
        grid=(N // _MB,),
        in_specs=[
            pl.BlockSpec((_MB, D), lambda i: (i, 0)),
            pl.BlockSpec((D, D), lambda i: (0, 0)),
        ],
        out_specs=pl.BlockSpec((_MB, D), lambda i: (i, 0)),
        out_shape=jax.ShapeDtypeStruct((N, D), jnp.float32),
    )(x, w)


def _combine_mm_body(p_ref, b_ref, w_ref, o_ref):
    x = jnp.maximum(p_ref[0] + p_ref[1] + b_ref[...], 0.0)
    o_ref[...] = jnp.dot(x, w_ref[...], preferred_element_type=jnp.float32)


@jax.jit
def _combine_mm(p, b, w):
    return pl.pallas_call(
        _combine_mm_body,
        grid=(N // _MB,),
        in_specs=[
            pl.BlockSpec((NC, _MB, D), lambda i: (0, i, 0)),
            pl.BlockSpec((1, D), lambda i: (0, 0)),
            pl.BlockSpec((D, D), lambda i: (0, 0)),
        ],
        out_specs=pl.BlockSpec((_MB, D), lambda i: (i, 0)),
        out_shape=jax.ShapeDtypeStruct((N, D), jnp.float32),
    )(p, b, w)


def _head_body(p_ref, b2_ref, wh1_ref, bh1_ref, wh2_ref, bh2_ref,
               wc_ref, bc_ref, o_ref):
    x = jnp.maximum(p_ref[0] + p_ref[1] + b2_ref[...], 0.0)
    h = jnp.maximum(
        jnp.dot(x, wh1_ref[...], preferred_element_type=jnp.float32)
        + bh1_ref[...], 0.0)
    h = jnp.maximum(
        jnp.dot(h, wh2_ref[...], preferred_element_type=jnp.float32)
        + bh2_ref[...], 0.0)
    av = (jnp.dot(h, wc_ref[...], preferred_element_type=jnp.float32)
          + bc_ref[...])
    col = lax.broadcasted_iota(jnp.int32, av.shape, 1)
    adv_sum = jnp.sum(jnp.where(col < 5, av, 0.0), axis=1, keepdims=True)
    val = jnp.sum(jnp.where(col == 5, av, 0.0), axis=1, keepdims=True)
    o_ref[...] = val + av - adv_sum * (1.0 / 5.0)


@jax.jit
def _head(p, b2, wh1, bh1, wh2, bh2, wc, bc):
    return pl.pallas_call(
        _head_body,
        grid=(N // _MB,),
        in_specs=[
            pl.BlockSpec((NC, _MB, D), lambda i: (0, i, 0)),
            pl.BlockSpec((1, D), lambda i: (0, 0)),
            pl.BlockSpec((D, D), lambda i: (0, 0)),
            pl.BlockSpec((1, D), lambda i: (0, 0)),
            pl.BlockSpec((D, D), lambda i: (0, 0)),
            pl.BlockSpec((1, D), lambda i: (0, 0)),
            pl.BlockSpec((D, 8), lambda i: (0, 0)),
            pl.BlockSpec((1, 8), lambda i: (0, 0)),
        ],
        out_specs=pl.BlockSpec((_MB, 8), lambda i: (i, 0)),
        out_shape=jax.ShapeDtypeStruct((N, 8), jnp.float32),
    )(p, b2, wh1, bh1, wh2, bh2, wc, bc)


def kernel(edge_index, entity_embeddings, W1, b1, W2, b2,
           Wh1, bh1, Wh2, bh2, Wadv, badv, Wval, bval):
    src = edge_index[0]
    dst = edge_index[1]
    wc = jnp.concatenate([Wadv, Wval, jnp.zeros((D, 2), jnp.float32)], axis=1)
    bc = jnp.concatenate([badv, bval, jnp.zeros((2,), jnp.float32)])[None, :]

    m1 = _mm(entity_embeddings, W1)
    p1 = _agg(src, dst, m1)
    m2 = _combine_mm(p1, b1[None, :], W2)
    p2 = _agg(src, dst, m2)
    q8 = _head(p2, b2[None, :], Wh1, bh1[None, :], Wh2, bh2[None, :], wc, bc)
    return q8[:, :5]


# speedup vs baseline: 3.6079x; 1.0012x over previous
import jax
import jax.numpy as jnp
from jax import lax
from jax.experimental import pallas as pl
from jax.experimental.pallas import tpu as pltpu
from jax.experimental.pallas import tpu_sc as plsc

N = 10000
E = 320000
D = 128

NC = 2
NS = 16
NW = NC * NS
EPW = E // NW
CH = 128
NFULL = EPW // CH
REM = EPW - NFULL * CH
ST = 640
ST_LAST = N - 15 * ST

_MB = 1000


def _agg_body(src_hbm, dst_hbm, m_hbm, out_hbm,
              sv0, sv1, dv0, dv1, r0, r1, srcr, dstr, rowsr, acc,
              gs0, gs1, ss0, ss1, is0, is1):
    srcv = [sv0, sv1]
    dstv = [dv0, dv1]
    rows = [r0, r1]
    gsem = [gs0, gs1]
    ssem = [ss0, ss1]
    isem = [is0, is1]
    c = lax.axis_index("c")
    s = lax.axis_index("s")

    zero16 = jnp.zeros((16,), jnp.float32)

    def _zfill(i, carry):
        for j in range(8):
            r0[i, pl.ds(j * 16, 16)] = zero16
        return carry

    lax.fori_loop(0, CH, _zfill, 0)

    def _zfill_r(i, carry):
        for j in range(8):
            rowsr[i, pl.ds(j * 16, 16)] = zero16
        return carry

    lax.fori_loop(0, REM, _zfill_r, 0)
    ofs = pl.multiple_of(s * ST, 8)

    @pl.when(s < 15)
    def _():
        for k in range(ST // CH):
            pltpu.sync_copy(r0, acc.at[pl.ds(ofs + k * CH, CH), :])

    @pl.when(s == 15)
    def _():
        for k in range(3):
            pltpu.sync_copy(r0, acc.at[pl.ds(15 * ST + k * CH, CH), :])
        pltpu.sync_copy(rowsr, acc.at[pl.ds(15 * ST + 3 * CH, REM), :])

    plsc.subcore_barrier()

    ebase = c * (E // NC) + s * EPW

    def fetch_dst(i, b):
        base = pl.multiple_of(ebase + i * CH, 8)
        pltpu.sync_copy(dst_hbm.at[pl.ds(base, CH)], dstv[b])

    def wait_scatter(b):
        pltpu.make_async_copy(rows[b], acc.at[pl.ds(0, CH), :],
                              ssem[b]).wait()

    def gather(b):
        pltpu.async_copy(m_hbm.at[srcv[b]], rows[b], gsem[b])

    def wait_gather(b):
        pltpu.make_async_copy(m_hbm.at[pl.ds(0, CH), :], rows[b],
                              gsem[b]).wait()

    def scatter(b):
        pltpu.async_copy(rows[b], acc.at[dstv[b]], ssem[b], add=True)

    def prefetch_src(i, b):
        base = pl.multiple_of(ebase + i * CH, 8)
        pltpu.async_copy(src_hbm.at[pl.ds(base, CH)], srcv[b], isem[b])

    def wait_src(b):
        pltpu.make_async_copy(src_hbm.at[pl.ds(0, CH)], srcv[b],
                              isem[b]).wait()

    prefetch_src(0, 0)
    prefetch_src(1, 1)
    wait_src(0)
    fetch_dst(0, 0)
    gather(0)
    wait_src(1)
    fetch_dst(1, 1)
    gather(1)
    wait_gather(0)
    scatter(0)
    prefetch_src(2, 0)

    def _do(i, b):
        wait_src(b)
        wait_scatter(b)
        fetch_dst(i, b)
        gather(b)
        wait_gather(1 - b)
        scatter(1 - b)

        @pl.when(i + 1 < NFULL)
        def _():
            prefetch_src(i + 1, 1 - b)

    def _pair(j, carry):
        _do(2 * j, 0)
        _do(2 * j + 1, 1)
        return carry

    lax.fori_loop(1, NFULL // 2, _pair, 0)
    wait_gather(1)
    scatter(1)
    for b in range(2):
        wait_scatter(b)

    base = pl.multiple_of(ebase + NFULL * CH, 8)
    pltpu.sync_copy(src_hbm.at[pl.ds(base, REM)], srcr)
    pltpu.sync_copy(dst_hbm.at[pl.ds(base, REM)], dstr)
    pltpu.async_copy(m_hbm.at[srcr], rowsr, gs0).wait()
    pltpu.sync_copy(rowsr, acc.at[dstr], add=True)

    plsc.subcore_barrier()

    @pl.when(s < 15)
    def _():
        pltpu.sync_copy(acc.at[pl.ds(ofs, ST), :],
                        out_hbm.at[c, pl.ds(ofs, ST), :])

    @pl.when(s == 15)
    def _():
        pltpu.sync_copy(acc.at[pl.ds(15 * ST, ST_LAST), :],
                        out_hbm.at[c, pl.ds(15 * ST, ST_LAST), :])


@jax.jit
def _agg(src, dst, m):
    mesh = plsc.VectorSubcoreMesh(core_axis_name="c", subcore_axis_name="s")
    return pl.kernel(
        _agg_body,
        out_type=jax.ShapeDtypeStruct((NC, N, D), jnp.float32),
        mesh=mesh,
        scratch_types=[
            pltpu.VMEM((CH,), jnp.int32),
            pltpu.VMEM((CH,), jnp.int32),
            pltpu.VMEM((CH,), jnp.int32),
            pltpu.VMEM((CH,), jnp.int32),
            pltpu.VMEM((CH, D), jnp.float32),
            pltpu.VMEM((CH, D), jnp.float32),
            pltpu.VMEM((REM,), jnp.int32),
            pltpu.VMEM((REM,), jnp.int32),
            pltpu.VMEM((REM, D), jnp.float32),
            pltpu.VMEM_SHARED((N, D), jnp.float32),
        ] + [pltpu.SemaphoreType.DMA] * 6,
    )(src, dst, m)


def _mm_body(x_ref, w_ref, o_ref):
    o_ref[...] = jnp.dot(x_ref[...], w_ref[...],
                         preferred_element_type=jnp.float32)


@jax.jit
def _mm(x, w):
    return pl.pallas_call(
        _mm_body,
        grid=(N // _MB,),
        in_specs=[
            pl.BlockSpec((_MB, D), lambda i: (i, 0)),
            pl.BlockSpec((D, D), lambda i: (0, 0)),
        ],
        out_specs=pl.BlockSpec((_MB, D), lambda i: (i, 0)),
        out_shape=jax.ShapeDtypeStruct((N, D), jnp.float32),
    )(x, w)


def _combine_mm_body(p_ref, b_ref, w_ref, o_ref):
    x = jnp.maximum(p_ref[0] + p_ref[1] + b_ref[...], 0.0)
    o_ref[...] = jnp.dot(x, w_ref[...], preferred_element_type=jnp.float32)


@jax.jit
def _combine_mm(p, b, w):
    return pl.pallas_call(
        _combine_mm_body,
        grid=(N // _MB,),
        in_specs=[
            pl.BlockSpec((NC, _MB, D), lambda i: (0, i, 0)),
            pl.BlockSpec((1, D), lambda i: (0, 0)),
            pl.BlockSpec((D, D), lambda i: (0, 0)),
        ],
        out_specs=pl.BlockSpec((_MB, D), lambda i: (i, 0)),
        out_shape=jax.ShapeDtypeStruct((N, D), jnp.float32),
    )(p, b, w)


def _head_body(p_ref, b2_ref, wh1_ref, bh1_ref, wh2_ref, bh2_ref,
               wc_ref, bc_ref, o_ref):
    x = jnp.maximum(p_ref[0] + p_ref[1] + b2_ref[...], 0.0)
    h = jnp.maximum(
        jnp.dot(x, wh1_ref[...], preferred_element_type=jnp.float32)
        + bh1_ref[...], 0.0)
    h = jnp.maximum(
        jnp.dot(h, wh2_ref[...], preferred_element_type=jnp.float32)
        + bh2_ref[...], 0.0)
    av = (jnp.dot(h, wc_ref[...], preferred_element_type=jnp.float32)
          + bc_ref[...])
    col = lax.broadcasted_iota(jnp.int32, av.shape, 1)
    adv_sum = jnp.sum(jnp.where(col < 5, av, 0.0), axis=1, keepdims=True)
    val = jnp.sum(jnp.where(col == 5, av, 0.0), axis=1, keepdims=True)
    o_ref[...] = val + av - adv_sum * (1.0 / 5.0)


@jax.jit
def _head(p, b2, wh1, bh1, wh2, bh2, wc, bc):
    return pl.pallas_call(
        _head_body,
        grid=(N // _MB,),
        in_specs=[
            pl.BlockSpec((NC, _MB, D), lambda i: (0, i, 0)),
            pl.BlockSpec((1, D), lambda i: (0, 0)),
            pl.BlockSpec((D, D), lambda i: (0, 0)),
            pl.BlockSpec((1, D), lambda i: (0, 0)),
            pl.BlockSpec((D, D), lambda i: (0, 0)),
            pl.BlockSpec((1, D), lambda i: (0, 0)),
            pl.BlockSpec((D, 8), lambda i: (0, 0)),
            pl.BlockSpec((1, 8), lambda i: (0, 0)),
        ],
        out_specs=pl.BlockSpec((_MB, 8), lambda i: (i, 0)),
        out_shape=jax.ShapeDtypeStruct((N, 8), jnp.float32),
    )(p, b2, wh1, bh1, wh2, bh2, wc, bc)


def kernel(edge_index, entity_embeddings, W1, b1, W2, b2,
           Wh1, bh1, Wh2, bh2, Wadv, badv, Wval, bval):
    src = edge_index[0]
    dst = edge_index[1]
    wc = jnp.concatenate([Wadv, Wval, jnp.zeros((D, 2), jnp.float32)], axis=1)
    bc = jnp.concatenate([badv, bval, jnp.zeros((2,), jnp.float32)])[None, :]

    m1 = _mm(entity_embeddings, W1)
    p1 = _agg(src, dst, m1)
    m2 = _combine_mm(p1, b1[None, :], W2)
    p2 = _agg(src, dst, m2)
    q8 = _head(p2, b2[None, :], Wh1, bh1[None, :], Wh2, bh2[None, :], wc, bc)
    return q8[:, :5]
